# Initial kernel scaffold; baseline (speedup 1.0000x reference)
import jax
import jax.numpy as jnp
from jax.experimental import pallas as pl


def kernel(ent_emb, attr_emb, rel_emb, rgcn_weight, dec_W, dec_b, time_emb, edge_index, edge_type, batch_data):
    raise NotImplementedError("write your pallas kernel here")


# speedup vs baseline: 8.0288x; 8.0288x over previous
import functools
import math

import jax
import jax.numpy as jnp
from jax import lax
from jax.experimental import pallas as pl
from jax.experimental.pallas import tpu as pltpu
from jax.experimental.pallas import tpu_sc as plsc

F32 = jnp.float32

N_ENT = 8000
N_ATTRV = 2000
N_NODES = N_ENT + N_ATTRV
N_TYPES = 16
HID = 200
D = 208
NPAD = 10240
N_EDGES = 320000
BATCH = 1024
NREL = 230
NTIME = 365
TDIV = 24
AGG_ROWS = 8016
ROWS_PER_SUB = AGG_ROWS // 16
RRELU = (1.0 / 8.0 + 1.0 / 3.0) / 2.0

NW = 32
EPW = N_EDGES // NW
CH = 40
SB = 2000
NCH_SB = SB // CH
NSB = EPW // SB
DROWS_PER_TILE = EPW // CH
BPW = BATCH // NW

def _sc_mesh():
    return plsc.VectorSubcoreMesh(core_axis_name="c", subcore_axis_name="s",
                                  num_cores=2, num_subcores=16)


NODES_PER_TILE = 320
RCH = 80
HVALID = RCH * HID


def _table_body(ent_h, attr_h, w_h, table_out, hflat, ob0, ob1, wbuf,
                sem_h, sem_o):
    c = lax.axis_index("c")
    s = lax.axis_index("s")
    wid = s * 2 + c
    nbase = wid * NODES_PER_TILE

    pltpu.sync_copy(w_h, wbuf)
    hflat[pl.ds(HVALID, 16)] = jnp.zeros((16,), F32)
    iota = lax.iota(jnp.int32, 16)
    cvec = jnp.where(iota == 15, 1.0, 0.0).astype(F32)

    nchunks = jnp.where(wid == 31, 1, NODES_PER_TILE // RCH)

    def chunk(cb, carry):
        node0 = nbase + cb * RCH

        @pl.when(wid < 25)
        def _():
            pltpu.sync_copy(ent_h.at[pl.ds(node0 * HID, HVALID)],
                            hflat.at[pl.ds(0, HVALID)])

        @pl.when(wid >= 25)
        def _():
            pltpu.sync_copy(attr_h.at[pl.ds(node0 * HID - N_ENT * HID, HVALID)],
                            hflat.at[pl.ds(0, HVALID)])

        for t in range(N_TYPES):
            ob = ob0 if t % 2 == 0 else ob1
            if t >= 2:
                pltpu.make_async_copy(
                    ob, table_out.at[pl.ds((t - 2) * NPAD + node0, RCH)],
                    sem_o).wait()

            wv = [wbuf[t, pl.ds(j * 16, 16)] for j in range(13)]
            wv12 = wv[12]

            @plsc.parallel_loop(0, RCH, 1, unroll=2)
            def _rows(r):
                for j in range(12):
                    hv = hflat[pl.ds(r * HID + j * 16, 16)]
                    ob[r, pl.ds(j * 16, 16)] = hv * wv[j]
                hv = hflat[pl.ds(r * HID + 192, 16)]
                ob[r, pl.ds(192, 16)] = hv * wv12 + cvec
            pltpu.async_copy(ob, table_out.at[pl.ds(t * NPAD + node0, RCH)],
                             sem_o)
        pltpu.make_async_copy(
            ob0, table_out.at[pl.ds((N_TYPES - 2) * NPAD + node0, RCH)],
            sem_o).wait()
        pltpu.make_async_copy(
            ob1, table_out.at[pl.ds((N_TYPES - 1) * NPAD + node0, RCH)],
            sem_o).wait()
        return carry

    lax.fori_loop(0, nchunks, chunk, 0)


def _build_table(ent_flat, attr_flat, wb0):
    k = functools.partial(
        pl.kernel,
        out_type=jax.ShapeDtypeStruct((N_TYPES * NPAD, D), F32),
        mesh=_sc_mesh(),
        scratch_types=[
            pltpu.VMEM((HVALID + 16,), F32),
            pltpu.VMEM((RCH, D), F32),
            pltpu.VMEM((RCH, D), F32),
            pltpu.VMEM((N_TYPES, D), F32),
            pltpu.SemaphoreType.DMA,
            pltpu.SemaphoreType.DMA,
        ],
        compiler_params=pltpu.CompilerParams(use_tc_tiling_on_sc=False),
    )(_table_body)
    return k(ent_flat, attr_flat, wb0)


def _index_body(src_ref, dst_ref, typ_ref, gidx_ref, didx_ref):
    gidx_ref[...] = typ_ref[...] * NPAD + src_ref[...]
    didx_ref[...] = jnp.minimum(dst_ref[...], N_ENT)


def _build_indices(src, dst, typ):
    src2 = src.reshape(2500, 128)
    dst2 = dst.reshape(2500, 128)
    typ2 = typ.reshape(2500, 128)
    gidx, didx = pl.pallas_call(
        _index_body,
        out_shape=(jax.ShapeDtypeStruct((2500, 128), jnp.int32),
                   jax.ShapeDtypeStruct((2500, 128), jnp.int32)),
    )(src2, dst2, typ2)
    return gidx.reshape(N_EDGES), didx.reshape(N_EDGES // CH, CH)


def _edge_body(table, gidx_h, didx_h, out_h,
               gb0, gb1, db0, db1, rows0, rows1,
               agg_sh, sem_z, sem_s, sem0, sem1):
    c = lax.axis_index("c")
    s = lax.axis_index("s")
    wid = s * 2 + c
    ebase = wid * EPW
    dbase = wid * DROWS_PER_TILE

    def stage_start(sb, gb, db):
        pltpu.async_copy(gidx_h.at[pl.ds(ebase + sb * SB, SB)], gb, sem_s)
        pltpu.async_copy(didx_h.at[pl.ds(dbase + sb * NCH_SB, NCH_SB)], db, sem_s)

    def stage_wait(sb, gb, db):
        pltpu.make_async_copy(gidx_h.at[pl.ds(ebase + sb * SB, SB)], gb, sem_s).wait()
        pltpu.make_async_copy(didx_h.at[pl.ds(dbase + sb * NCH_SB, NCH_SB)], db, sem_s).wait()

    stage_start(0, gb0, db0)

    zv = jnp.zeros((16,), F32)

    def zrow(r, carry):
        for j in range(D // 16):
            rows0[r, pl.ds(j * 16, 16)] = zv
        return carry

    lax.fori_loop(0, CH, zrow, 0)
    nfull = ROWS_PER_SUB // CH
    ztail = ROWS_PER_SUB - nfull * CH
    abase = s * ROWS_PER_SUB
    for k in range(nfull):
        pltpu.async_copy(rows0, agg_sh.at[pl.ds(abase + k * CH, CH)], sem_z)
    pltpu.async_copy(rows0.at[pl.ds(0, ztail)],
                     agg_sh.at[pl.ds(abase + nfull * CH, ztail)], sem_z)
    for k in range(nfull):
        pltpu.make_async_copy(rows0, agg_sh.at[pl.ds(abase + k * CH, CH)],
                              sem_z).wait()
    pltpu.make_async_copy(rows0.at[pl.ds(0, ztail)],
                          agg_sh.at[pl.ds(abase + nfull * CH, ztail)],
                          sem_z).wait()
    plsc.subcore_barrier()

    for sb in range(NSB):
        gb, db = (gb0, db0) if sb % 2 == 0 else (gb1, db1)
        ngb, ndb = (gb1, db1) if sb % 2 == 0 else (gb0, db0)
        stage_wait(sb, gb, db)
        if sb + 1 < NSB:
            stage_start(sb + 1, ngb, ndb)

        def g_start(ci, rbuf, sem):
            pltpu.async_copy(table.at[gb.at[pl.ds(ci * CH, CH)]], rbuf, sem)

        def g_wait(ci, rbuf, sem):
            pltpu.make_async_copy(table.at[gb.at[pl.ds(ci * CH, CH)]],
                                  rbuf, sem).wait()

        def scat(ci, rbuf):
            pltpu.sync_copy(rbuf, agg_sh.at[db.at[ci]], add=True)

        g_start(0, rows0, sem0)

        def pair(k, carry):
            c0 = 2 * k
            g_start(c0 + 1, rows1, sem1)
            g_wait(c0, rows0, sem0)
            scat(c0, rows0)
            g_start(c0 + 2, rows0, sem0)
            g_wait(c0 + 1, rows1, sem1)
            scat(c0 + 1, rows1)
            return carry

        lax.fori_loop(0, NCH_SB // 2 - 1, pair, 0)
        c0 = NCH_SB - 2
        g_start(c0 + 1, rows1, sem1)
        g_wait(c0, rows0, sem0)
        scat(c0, rows0)
        g_wait(c0 + 1, rows1, sem1)
        scat(c0 + 1, rows1)

    plsc.subcore_barrier()
    pltpu.sync_copy(agg_sh.at[pl.ds(s * ROWS_PER_SUB, ROWS_PER_SUB)],
                    out_h.at[c, pl.ds(s * ROWS_PER_SUB, ROWS_PER_SUB)])


def _edge_kernel(*args):
    k = functools.partial(
        pl.kernel,
        out_type=jax.ShapeDtypeStruct((2, AGG_ROWS, D), F32),
        mesh=_sc_mesh(),
        scratch_types=[
            pltpu.VMEM((SB,), jnp.int32),
            pltpu.VMEM((SB,), jnp.int32),
            pltpu.VMEM((NCH_SB, CH), jnp.int32),
            pltpu.VMEM((NCH_SB, CH), jnp.int32),
            pltpu.VMEM((CH, D), F32),
            pltpu.VMEM((CH, D), F32),
            pltpu.VMEM_SHARED((AGG_ROWS, D), F32),
            pltpu.SemaphoreType.DMA,
            pltpu.SemaphoreType.DMA,
            pltpu.SemaphoreType.DMA,
            pltpu.SemaphoreType.DMA,
        ],
        compiler_params=pltpu.CompilerParams(use_tc_tiling_on_sc=False),
    )(_edge_body)
    return k(*args)


def _static_body(agg_ref, out_ref):
    a = agg_ref[0] + agg_ref[1]
    deg = a[:, D - 1:D]
    norm = jnp.where(deg > 0.0, 1.0 / jnp.maximum(deg, 1.0), 0.0)
    s = a * norm
    out_ref[...] = jnp.where(s >= 0.0, s, s * RRELU)


def _build_static(aggs):
    return pl.pallas_call(
        _static_body,
        grid=(8,),
        in_specs=[pl.BlockSpec((2, 1000, D), lambda j: (0, j, 0))],
        out_specs=pl.BlockSpec((1000, D), lambda j: (j, 0)),
        out_shape=jax.ShapeDtypeStruct((N_ENT, D), F32),
    )(aggs)


def _gather_body(static_h, rel_h, time_h, eidx_h, ridx_h, bd3_h,
                 ent_out, rel_out, tim_out,
                 ibuf, tibuf, erows, rrows, trows, sem):
    c = lax.axis_index("c")
    s = lax.axis_index("s")
    wid = s * 2 + c
    base = wid * BPW

    pltpu.sync_copy(eidx_h.at[pl.ds(base, BPW)], ibuf)
    pltpu.async_copy(static_h.at[ibuf], erows, sem).wait()
    pltpu.sync_copy(erows, ent_out.at[pl.ds(base, BPW)])

    pltpu.sync_copy(ridx_h.at[pl.ds(base, BPW)], ibuf)
    pltpu.async_copy(rel_h.at[ibuf], rrows, sem).wait()
    pltpu.sync_copy(rrows, rel_out.at[pl.ds(base, BPW)])

    pltpu.sync_copy(bd3_h.at[pl.ds(base, BPW)], ibuf)
    tdiv = jnp.full((16,), TDIV, jnp.int32)
    for j in range(BPW // 16):
        sl = pl.ds(j * 16, 16)
        tibuf[sl] = lax.div(ibuf[sl], tdiv)
    pltpu.async_copy(time_h.at[tibuf], trows, sem).wait()
    pltpu.sync_copy(trows, tim_out.at[pl.ds(base, BPW)])


def _gather_kernel(*args):
    k = functools.partial(
        pl.kernel,
        out_type=(jax.ShapeDtypeStruct((BATCH, D), F32),
                  jax.ShapeDtypeStruct((BATCH, D), F32),
                  jax.ShapeDtypeStruct((BATCH, 16), F32)),
        mesh=_sc_mesh(),
        scratch_types=[
            pltpu.VMEM((BPW,), jnp.int32),
            pltpu.VMEM((BPW,), jnp.int32),
            pltpu.VMEM((BPW, D), F32),
            pltpu.VMEM((BPW, D), F32),
            pltpu.VMEM((BPW, 16), F32),
            pltpu.SemaphoreType.DMA,
        ],
        compiler_params=pltpu.CompilerParams(use_tc_tiling_on_sc=False),
    )(_gather_body)
    return k(*args)


def _decode_body(ent_ref, rel_ref, tim_ref, wa_ref, wb_ref, wcb_ref, s_ref,
                 out_ref):
    ent = jnp.tanh(ent_ref[...])
    x = lax.dot_general(ent, wa_ref[...], (((1,), (0,)), ((), ())),
                        preferred_element_type=F32)
    x = x + lax.dot_general(rel_ref[...], wb_ref[...],
                            (((1,), (0,)), ((), ())),
                            preferred_element_type=F32)
    x = x + tim_ref[...][:, 0:1] * wcb_ref[0:1, :]
    x = x + wcb_ref[1:2, :]
    x = jnp.maximum(x, 0.0)
    out_ref[...] = lax.dot_general(x, s_ref[...],
                                   (((1,), (1,)), ((), ())),
                                   preferred_element_type=F32)


def _decode(ent_rows, rel_rows, tim_rows, wa, wb, wcb, static):
    return pl.pallas_call(
        _decode_body,
        out_shape=jax.ShapeDtypeStruct((BATCH, N_ENT), F32),
    )(ent_rows, rel_rows, tim_rows, wa, wb, wcb, static)


def kernel(ent_emb, attr_emb, rel_emb, rgcn_weight, dec_W, dec_b, time_emb,
           edge_index, edge_type, batch_data):
    ent_flat = ent_emb.reshape(N_ENT * HID)
    attr_flat = attr_emb.reshape(N_ATTRV * HID)
    wb0 = jnp.zeros((N_TYPES, D), F32).at[:, :HID].set(rgcn_weight)
    rel_pad = jnp.zeros((NREL, D), F32).at[:, :HID].set(rel_emb)
    time_pad = jnp.zeros((NTIME, 16), F32).at[:, 0:1].set(time_emb)
    wa = jnp.zeros((D, D), F32).at[:HID, :HID].set(dec_W[:HID])
    wb = jnp.zeros((D, D), F32).at[:HID, :HID].set(dec_W[HID:2 * HID])
    wcb = (jnp.zeros((2, D), F32)
           .at[0, :HID].set(dec_W[2 * HID])
           .at[1, :HID].set(dec_b))

    src = edge_index[0]
    dst = edge_index[1]
    eidx = batch_data[:, 0]
    ridx = batch_data[:, 1]
    bd3 = batch_data[:, 3]

    table = _build_table(ent_flat, attr_flat, wb0)
    gidx, didx = _build_indices(src, dst, edge_type)
    aggs = _edge_kernel(table, gidx, didx)
    static = _build_static(aggs)
    ent_rows, rel_rows, tim_rows = _gather_kernel(
        static, rel_pad, time_pad, eidx, ridx, bd3)
    return _decode(ent_rows, rel_rows, tim_rows, wa, wb, wcb, static)
